# baseline (speedup 1.0000x reference)
import functools

import jax
import jax.numpy as jnp
from jax import lax
from jax.experimental import pallas as pl
from jax.experimental.pallas import tpu as pltpu
from jax.experimental.pallas import tpu_sc as plsc

B = 16384
H = 128
EPS = 1e-12

NC = 2
NS = 16
NW = NC * NS
ROWS_PER_W = B // NW
CHUNK = 128
NCHUNK = ROWS_PER_W // CHUNK
L = 16
VPR = H // L


def _rsqrt_vec(v):
    i = lax.bitcast_convert_type(v, jnp.int32)
    i = jnp.int32(0x5F3759DF) - lax.shift_right_logical(i, 1)
    y = lax.bitcast_convert_type(i, jnp.float32)
    half = v * 0.5
    for _ in range(3):
        y = y * (1.5 - half * y * y)
    return y


def _allreduce_sum(v, lanes):
    for k in (8, 4, 2, 1):
        perm = v.at[lanes ^ k].get(mode="promise_in_bounds",
                                   unique_indices=True)
        v = v + perm
    return v


_mesh = plsc.VectorSubcoreMesh(core_axis_name="c", subcore_axis_name="s")


@functools.partial(
    pl.kernel,
    mesh=_mesh,
    out_type=jax.ShapeDtypeStruct((B, H), jnp.float32),
    scratch_types=[
        pltpu.VMEM((4 * ROWS_PER_W,), jnp.int32),
        pltpu.VMEM((4, ROWS_PER_W), jnp.int32),
        pltpu.VMEM((CHUNK, H), jnp.float32),
        pltpu.VMEM((CHUNK, H), jnp.float32),
        pltpu.VMEM((CHUNK, H), jnp.float32),
        pltpu.VMEM((CHUNK, H), jnp.float32),
        pltpu.VMEM((H,), jnp.float32),
        pltpu.VMEM((H,), jnp.float32),
        pltpu.SemaphoreType.DMA,
        pltpu.SemaphoreType.DMA,
        pltpu.SemaphoreType.DMA,
        pltpu.SemaphoreType.DMA,
    ],
)
def _embed_ln(x_flat, syn, pos, sen, lem, gamma, beta, out,
              xblk, idx_v, ga, gb, oa, ob, g_v, be_v,
              sem_ga, sem_gb, sem_oa, sem_ob):
    wid = lax.axis_index("s") * NC + lax.axis_index("c")
    base = wid * ROWS_PER_W
    lanes = lax.iota(jnp.int32, L)
    pltpu.sync_copy(gamma, g_v)
    pltpu.sync_copy(beta, be_v)
    pltpu.sync_copy(x_flat.at[pl.ds(base * 4, 4 * ROWS_PER_W)], xblk)

    perm_base = (lanes & 3) * 4
    quarter = lax.shift_right_logical(lanes, 2)
    qmask = [quarter == q for q in range(3)]

    def deint_body(g, carry):
        g64 = g * 64
        vs = [xblk[pl.ds(g64 + i * L, L)] for i in range(4)]
        for t in range(4):
            pt = perm_base + t
            qs = [v.at[pt].get(mode="promise_in_bounds") for v in vs]
            w = jnp.where(qmask[0], qs[0],
                          jnp.where(qmask[1], qs[1],
                                    jnp.where(qmask[2], qs[2], qs[3])))
            idx_v[t, pl.ds(g * L, L)] = w
        return carry

    tables = (syn, pos, sen, lem)
    gbufs = (ga, gb)
    obufs = (oa, ob)
    gsems = (sem_ga, sem_gb)
    osems = (sem_oa, sem_ob)

    zero = jnp.zeros((L,), jnp.float32)

    def zero_buf(buf):
        def zbody(r, carry):
            for j in range(2 * VPR):
                buf[2 * r + j // VPR, pl.ds((j % VPR) * L, L)] = zero
            return carry
        lax.fori_loop(0, CHUNK // 2, zbody, 0)

    def fire_gathers(c):
        p = c % 2
        return [
            pltpu.async_copy(
                tables[t].at[idx_v.at[t, pl.ds(c * CHUNK, CHUNK)]],
                gbufs[p], gsems[p], add=True)
            for t in range(4)
        ]

    gvs = [g_v[pl.ds(j * L, L)] for j in range(VPR)]
    bevs = [be_v[pl.ds(j * L, L)] for j in range(VPR)]

    def compute_chunk(gbuf, obuf, rezero):
        def row_body(r2, carry):
            for rr in range(2):
                r = 2 * r2 + rr
                accs = [gbuf[r, pl.ds(j * L, L)] for j in range(VPR)]
                if rezero:
                    for j in range(VPR):
                        gbuf[r, pl.ds(j * L, L)] = zero
                s = accs[0]
                for j in range(1, VPR):
                    s = s + accs[j]
                sq = accs[0] * accs[0]
                for j in range(1, VPR):
                    sq = sq + accs[j] * accs[j]
                s = _allreduce_sum(s, lanes)
                sq = _allreduce_sum(sq, lanes)
                mean = s * (1.0 / H)
                var = sq * (1.0 / H) - mean * mean
                rinv = _rsqrt_vec(var + EPS)
                t0 = mean * rinv
                for j in range(VPR):
                    obuf[r, pl.ds(j * L, L)] = (
                        (accs[j] * rinv - t0) * gvs[j] + bevs[j])
            return carry
        lax.fori_loop(0, CHUNK // 2, row_body, 0)

    gpc = CHUNK // L
    zero_buf(ga)
    lax.fori_loop(0, gpc, deint_body, 0)
    g_copies = {0: fire_gathers(0)}
    zero_buf(gb)
    lax.fori_loop(gpc, ROWS_PER_W // L, deint_body, 0)
    g_copies[1] = fire_gathers(1)
    o_copies = {}

    for c in range(NCHUNK):
        p = c % 2
        for cp in g_copies.pop(c):
            cp.wait()
        if c >= 2:
            o_copies.pop(c - 2).wait()
        compute_chunk(gbufs[p], obufs[p], rezero=(c + 2 < NCHUNK))
        if c + 2 < NCHUNK:
            g_copies[c + 2] = fire_gathers(c + 2)
        o_copies[c] = pltpu.async_copy(
            obufs[p], out.at[pl.ds(base + c * CHUNK, CHUNK)], osems[p])

    for c in (NCHUNK - 2, NCHUNK - 1):
        o_copies.pop(c).wait()


def kernel(x, syn_table, lemma_table, pos_table, sense_table, gamma, beta):
    x_flat = x.reshape(-1)
    return _embed_ln(x_flat, syn_table, pos_table, sense_table, lemma_table,
                     gamma, beta)


# speedup vs baseline: 1.2495x; 1.2495x over previous
import functools

import jax
import jax.numpy as jnp
from jax import lax
from jax.experimental import pallas as pl
from jax.experimental.pallas import tpu as pltpu
from jax.experimental.pallas import tpu_sc as plsc

B = 16384
H = 128
EPS = 1e-12

NC = 2
NS = 16
NW = NC * NS
ROWS_PER_W = B // NW
CHUNK = 128
NCHUNK = ROWS_PER_W // CHUNK
L = 16
VPR = H // L


def _rsqrt_vec(v):
    i = lax.bitcast_convert_type(v, jnp.int32)
    i = jnp.int32(0x5F3759DF) - lax.shift_right_logical(i, 1)
    y = lax.bitcast_convert_type(i, jnp.float32)
    half = v * 0.5
    for _ in range(3):
        y = y * (1.5 - half * y * y)
    return y


def _allreduce_sum(v, lanes):
    for k in (8, 4, 2, 1):
        perm = v.at[lanes ^ k].get(mode="promise_in_bounds",
                                   unique_indices=True)
        v = v + perm
    return v


_mesh = plsc.VectorSubcoreMesh(core_axis_name="c", subcore_axis_name="s")


@functools.partial(
    pl.kernel,
    mesh=_mesh,
    out_type=jax.ShapeDtypeStruct((B, H), jnp.float32),
    scratch_types=[
        pltpu.VMEM((4, ROWS_PER_W), jnp.int32),
        pltpu.VMEM((CHUNK, H), jnp.float32),
        pltpu.VMEM((CHUNK, H), jnp.float32),
        pltpu.VMEM((CHUNK, H), jnp.float32),
        pltpu.VMEM((CHUNK, H), jnp.float32),
        pltpu.VMEM((H,), jnp.float32),
        pltpu.VMEM((H,), jnp.float32),
        pltpu.SemaphoreType.DMA,
        pltpu.SemaphoreType.DMA,
        pltpu.SemaphoreType.DMA,
        pltpu.SemaphoreType.DMA,
    ],
)
def _embed_ln(xT, syn, pos, sen, lem, gamma, beta, out,
              idx_v, ga, gb, oa, ob, g_v, be_v,
              sem_ga, sem_gb, sem_oa, sem_ob):
    wid = lax.axis_index("s") * NC + lax.axis_index("c")
    base = wid * ROWS_PER_W
    lanes = lax.iota(jnp.int32, L)
    pltpu.sync_copy(gamma, g_v)
    pltpu.sync_copy(beta, be_v)
    pltpu.sync_copy(xT.at[:, pl.ds(base, ROWS_PER_W)], idx_v)

    tables = (syn, pos, sen, lem)
    gbufs = (ga, gb)
    obufs = (oa, ob)
    gsems = (sem_ga, sem_gb)
    osems = (sem_oa, sem_ob)

    zero = jnp.zeros((L,), jnp.float32)

    def zero_buf(buf):
        def zbody(r, carry):
            for j in range(2 * VPR):
                buf[2 * r + j // VPR, pl.ds((j % VPR) * L, L)] = zero
            return carry
        lax.fori_loop(0, CHUNK // 2, zbody, 0)

    def fire_gathers(c):
        p = c % 2
        return [
            pltpu.async_copy(
                tables[t].at[idx_v.at[t, pl.ds(c * CHUNK, CHUNK)]],
                gbufs[p], gsems[p], add=True)
            for t in range(4)
        ]

    gvs = [g_v[pl.ds(j * L, L)] for j in range(VPR)]
    bevs = [be_v[pl.ds(j * L, L)] for j in range(VPR)]

    def compute_chunk(gbuf, obuf, rezero):
        def row_body(r2, carry):
            for rr in range(2):
                r = 2 * r2 + rr
                accs = [gbuf[r, pl.ds(j * L, L)] for j in range(VPR)]
                if rezero:
                    for j in range(VPR):
                        gbuf[r, pl.ds(j * L, L)] = zero
                s = accs[0]
                for j in range(1, VPR):
                    s = s + accs[j]
                sq = accs[0] * accs[0]
                for j in range(1, VPR):
                    sq = sq + accs[j] * accs[j]
                s = _allreduce_sum(s, lanes)
                sq = _allreduce_sum(sq, lanes)
                mean = s * (1.0 / H)
                var = sq * (1.0 / H) - mean * mean
                rinv = _rsqrt_vec(var + EPS)
                t0 = mean * rinv
                for j in range(VPR):
                    obuf[r, pl.ds(j * L, L)] = (
                        (accs[j] * rinv - t0) * gvs[j] + bevs[j])
            return carry
        lax.fori_loop(0, CHUNK // 2, row_body, 0)

    zero_buf(ga)
    g_copies = {0: fire_gathers(0)}
    zero_buf(gb)
    g_copies[1] = fire_gathers(1)
    o_copies = {}

    for c in range(NCHUNK):
        p = c % 2
        for cp in g_copies.pop(c):
            cp.wait()
        if c >= 2:
            o_copies.pop(c - 2).wait()
        compute_chunk(gbufs[p], obufs[p], rezero=(c + 2 < NCHUNK))
        if c + 2 < NCHUNK:
            g_copies[c + 2] = fire_gathers(c + 2)
        o_copies[c] = pltpu.async_copy(
            obufs[p], out.at[pl.ds(base + c * CHUNK, CHUNK)], osems[p])

    for c in (NCHUNK - 2, NCHUNK - 1):
        o_copies.pop(c).wait()


def kernel(x, syn_table, lemma_table, pos_table, sense_table, gamma, beta):
    xT = x.T
    return _embed_ln(xT, syn_table, pos_table, sense_table, lemma_table,
                     gamma, beta)


# speedup vs baseline: 1.2628x; 1.0107x over previous
import functools

import jax
import jax.numpy as jnp
from jax import lax
from jax.experimental import pallas as pl
from jax.experimental.pallas import tpu as pltpu
from jax.experimental.pallas import tpu_sc as plsc

B = 16384
H = 128
EPS = 1e-12

NC = 2
NS = 16
NW = NC * NS
ROWS_PER_W = B // NW
CHUNK = 128
NCHUNK = ROWS_PER_W // CHUNK
L = 16
VPR = H // L


def _rsqrt_vec(v):
    i = lax.bitcast_convert_type(v, jnp.int32)
    i = jnp.int32(0x5F3759DF) - lax.shift_right_logical(i, 1)
    y = lax.bitcast_convert_type(i, jnp.float32)
    half = v * 0.5
    for _ in range(3):
        y = y * (1.5 - half * y * y)
    return y


_mesh = plsc.VectorSubcoreMesh(core_axis_name="c", subcore_axis_name="s")


@functools.partial(
    pl.kernel,
    mesh=_mesh,
    out_type=jax.ShapeDtypeStruct((B, H), jnp.float32),
    scratch_types=[
        pltpu.VMEM((4, ROWS_PER_W), jnp.int32),
        pltpu.VMEM((CHUNK, H), jnp.float32),
        pltpu.VMEM((CHUNK, H), jnp.float32),
        pltpu.VMEM((CHUNK, H), jnp.float32),
        pltpu.VMEM((CHUNK, H), jnp.float32),
        pltpu.VMEM((H,), jnp.float32),
        pltpu.VMEM((H,), jnp.float32),
        pltpu.SemaphoreType.DMA,
        pltpu.SemaphoreType.DMA,
        pltpu.SemaphoreType.DMA,
        pltpu.SemaphoreType.DMA,
    ],
)
def _embed_ln(xT, syn, pos, sen, lem, gamma, beta, out,
              idx_v, ga, gb, oa, ob, g_v, be_v,
              sem_ga, sem_gb, sem_oa, sem_ob):
    wid = lax.axis_index("s") * NC + lax.axis_index("c")
    base = wid * ROWS_PER_W
    lanes = lax.iota(jnp.int32, L)
    pltpu.sync_copy(gamma, g_v)
    pltpu.sync_copy(beta, be_v)
    pltpu.sync_copy(xT.at[:, pl.ds(base, ROWS_PER_W)], idx_v)

    tables = (syn, pos, sen, lem)
    gbufs = (ga, gb)
    obufs = (oa, ob)
    gsems = (sem_ga, sem_gb)
    osems = (sem_oa, sem_ob)

    zero = jnp.zeros((L,), jnp.float32)

    def zero_buf(buf):
        def zbody(r, carry):
            for j in range(2 * VPR):
                buf[2 * r + j // VPR, pl.ds((j % VPR) * L, L)] = zero
            return carry
        lax.fori_loop(0, CHUNK // 2, zbody, 0)

    def fire_gathers(c):
        p = c % 2
        return [
            pltpu.async_copy(
                tables[t].at[idx_v.at[t, pl.ds(c * CHUNK, CHUNK)]],
                gbufs[p], gsems[p], add=True)
            for t in range(4)
        ]

    gvs = [g_v[pl.ds(j * L, L)] for j in range(VPR)]
    bevs = [be_v[pl.ds(j * L, L)] for j in range(VPR)]

    lo8 = lanes < 8
    swap8 = lanes ^ 8
    pick_a = lanes & 7
    pick_b = (lanes & 7) | 8

    def _perm(v, idx):
        return v.at[idx].get(mode="promise_in_bounds")

    def compute_chunk(gbuf, obuf, rezero):
        def row_body(r2, carry):
            ra = 2 * r2
            rb = ra + 1
            aa = [gbuf[ra, pl.ds(j * L, L)] for j in range(VPR)]
            ab = [gbuf[rb, pl.ds(j * L, L)] for j in range(VPR)]
            if rezero:
                for j in range(VPR):
                    gbuf[ra, pl.ds(j * L, L)] = zero
                    gbuf[rb, pl.ds(j * L, L)] = zero
            sa, sb = aa[0], ab[0]
            qa, qb = aa[0] * aa[0], ab[0] * ab[0]
            for j in range(1, VPR):
                sa = sa + aa[j]
                sb = sb + ab[j]
                qa = qa + aa[j] * aa[j]
                qb = qb + ab[j] * ab[j]
            sw = jnp.where(lo8, sa, _perm(sb, swap8))
            qw = jnp.where(lo8, qa, _perm(qb, swap8))
            for k in (4, 2, 1):
                sw = sw + _perm(sw, lanes ^ k)
                qw = qw + _perm(qw, lanes ^ k)
            mw = sw * (1.0 / H)
            varw = qw * (1.0 / H) - mw * mw
            rw = _rsqrt_vec(varw + EPS)
            tw = mw * rw
            rinv_a, rinv_b = _perm(rw, pick_a), _perm(rw, pick_b)
            t_a, t_b = _perm(tw, pick_a), _perm(tw, pick_b)
            for j in range(VPR):
                obuf[ra, pl.ds(j * L, L)] = (
                    (aa[j] * rinv_a - t_a) * gvs[j] + bevs[j])
                obuf[rb, pl.ds(j * L, L)] = (
                    (ab[j] * rinv_b - t_b) * gvs[j] + bevs[j])
            return carry
        lax.fori_loop(0, CHUNK // 2, row_body, 0)

    zero_buf(ga)
    g_copies = {0: fire_gathers(0)}
    zero_buf(gb)
    g_copies[1] = fire_gathers(1)
    o_copies = {}

    for c in range(NCHUNK):
        p = c % 2
        for cp in g_copies.pop(c):
            cp.wait()
        if c >= 2:
            o_copies.pop(c - 2).wait()
        compute_chunk(gbufs[p], obufs[p], rezero=(c + 2 < NCHUNK))
        if c + 2 < NCHUNK:
            g_copies[c + 2] = fire_gathers(c + 2)
        o_copies[c] = pltpu.async_copy(
            obufs[p], out.at[pl.ds(base + c * CHUNK, CHUNK)], osems[p])

    for c in (NCHUNK - 2, NCHUNK - 1):
        o_copies.pop(c).wait()


def kernel(x, syn_table, lemma_table, pos_table, sense_table, gamma, beta):
    xT = x.T
    return _embed_ln(xT, syn_table, pos_table, sense_table, lemma_table,
                     gamma, beta)


# speedup vs baseline: 1.2901x; 1.0216x over previous
import functools

import jax
import jax.numpy as jnp
from jax import lax
from jax.experimental import pallas as pl
from jax.experimental.pallas import tpu as pltpu
from jax.experimental.pallas import tpu_sc as plsc

B = 16384
H = 128
EPS = 1e-12

NC = 2
NS = 16
NW = NC * NS
ROWS_PER_W = B // NW
CHUNK = 128
NCHUNK = ROWS_PER_W // CHUNK
L = 16
VPR = H // L


def _rsqrt_vec(v):
    i = lax.bitcast_convert_type(v, jnp.int32)
    i = jnp.int32(0x5F3759DF) - lax.shift_right_logical(i, 1)
    y = lax.bitcast_convert_type(i, jnp.float32)
    half = v * 0.5
    for _ in range(3):
        y = y * (1.5 - half * y * y)
    return y


_mesh = plsc.VectorSubcoreMesh(core_axis_name="c", subcore_axis_name="s")


@functools.partial(
    pl.kernel,
    mesh=_mesh,
    out_type=jax.ShapeDtypeStruct((B, H), jnp.float32),
    scratch_types=[
        pltpu.VMEM((4, ROWS_PER_W), jnp.int32),
        pltpu.VMEM((CHUNK, H), jnp.float32),
        pltpu.VMEM((CHUNK, H), jnp.float32),
        pltpu.VMEM((CHUNK, H), jnp.float32),
        pltpu.VMEM((CHUNK, H), jnp.float32),
        pltpu.VMEM((H,), jnp.float32),
        pltpu.VMEM((H,), jnp.float32),
        pltpu.SemaphoreType.DMA,
        pltpu.SemaphoreType.DMA,
        pltpu.SemaphoreType.DMA,
        pltpu.SemaphoreType.DMA,
    ],
)
def _embed_ln(xT, syn, pos, sen, lem, gamma, beta, out,
              idx_v, ga, gb, oa, ob, g_v, be_v,
              sem_ga, sem_gb, sem_oa, sem_ob):
    wid = lax.axis_index("s") * NC + lax.axis_index("c")
    base = wid * ROWS_PER_W
    lanes = lax.iota(jnp.int32, L)
    cp_gamma = pltpu.async_copy(gamma, g_v, sem_oa)
    cp_beta = pltpu.async_copy(beta, be_v, sem_ob)
    cp_idx = pltpu.async_copy(
        xT.at[:, pl.ds(base, ROWS_PER_W)], idx_v, sem_ga)

    tables = (syn, pos, sen, lem)
    gbufs = (ga, gb)
    obufs = (oa, ob)
    gsems = (sem_ga, sem_gb)
    osems = (sem_oa, sem_ob)

    zero = jnp.zeros((L,), jnp.float32)

    def zero_buf(buf):
        def zbody(r, carry):
            for j in range(2 * VPR):
                buf[2 * r + j // VPR, pl.ds((j % VPR) * L, L)] = zero
            return carry
        lax.fori_loop(0, CHUNK // 2, zbody, 0)

    def fire_gathers(c):
        p = c % 2
        return [
            pltpu.async_copy(
                tables[t].at[idx_v.at[t, pl.ds(c * CHUNK, CHUNK)]],
                gbufs[p], gsems[p], add=True)
            for t in range(4)
        ]

    lo8 = lanes < 8
    swap8 = lanes ^ 8
    pick_a = lanes & 7
    pick_b = (lanes & 7) | 8

    def _perm(v, idx):
        return v.at[idx].get(mode="promise_in_bounds")

    def compute_chunk(gbuf, obuf, rezero):
        def row_body(r2, carry):
            ra = 2 * r2
            rb = ra + 1
            aa = [gbuf[ra, pl.ds(j * L, L)] for j in range(VPR)]
            ab = [gbuf[rb, pl.ds(j * L, L)] for j in range(VPR)]
            if rezero:
                for j in range(VPR):
                    gbuf[ra, pl.ds(j * L, L)] = zero
                    gbuf[rb, pl.ds(j * L, L)] = zero
            sa, sb = aa[0], ab[0]
            qa, qb = aa[0] * aa[0], ab[0] * ab[0]
            for j in range(1, VPR):
                sa = sa + aa[j]
                sb = sb + ab[j]
                qa = qa + aa[j] * aa[j]
                qb = qb + ab[j] * ab[j]
            sa = sa + _perm(sa, swap8)
            sb = sb + _perm(sb, swap8)
            qa = qa + _perm(qa, swap8)
            qb = qb + _perm(qb, swap8)
            sw = jnp.where(lo8, sa, _perm(sb, swap8))
            qw = jnp.where(lo8, qa, _perm(qb, swap8))
            for k in (4, 2, 1):
                sw = sw + _perm(sw, lanes ^ k)
                qw = qw + _perm(qw, lanes ^ k)
            mw = sw * (1.0 / H)
            varw = qw * (1.0 / H) - mw * mw
            rw = _rsqrt_vec(varw + EPS)
            tw = mw * rw
            rinv_a, rinv_b = _perm(rw, pick_a), _perm(rw, pick_b)
            t_a, t_b = _perm(tw, pick_a), _perm(tw, pick_b)
            for j in range(VPR):
                obuf[ra, pl.ds(j * L, L)] = (
                    (aa[j] * rinv_a - t_a) * gvs[j] + bevs[j])
                obuf[rb, pl.ds(j * L, L)] = (
                    (ab[j] * rinv_b - t_b) * gvs[j] + bevs[j])
            return carry
        lax.fori_loop(0, CHUNK // 2, row_body, 0)

    zero_buf(ga)
    zero_buf(gb)
    cp_idx.wait()
    g_copies = {0: fire_gathers(0), 1: fire_gathers(1)}
    cp_gamma.wait()
    cp_beta.wait()
    gvs = [g_v[pl.ds(j * L, L)] for j in range(VPR)]
    bevs = [be_v[pl.ds(j * L, L)] for j in range(VPR)]
    o_copies = {}

    for c in range(NCHUNK):
        p = c % 2
        for cp in g_copies.pop(c):
            cp.wait()
        if c >= 2:
            o_copies.pop(c - 2).wait()
        compute_chunk(gbufs[p], obufs[p], rezero=(c + 2 < NCHUNK))
        if c + 2 < NCHUNK:
            g_copies[c + 2] = fire_gathers(c + 2)
        o_copies[c] = pltpu.async_copy(
            obufs[p], out.at[pl.ds(base + c * CHUNK, CHUNK)], osems[p])

    for c in (NCHUNK - 2, NCHUNK - 1):
        o_copies.pop(c).wait()


def kernel(x, syn_table, lemma_table, pos_table, sense_table, gamma, beta):
    xT = x.T
    return _embed_ln(xT, syn_table, pos_table, sense_table, lemma_table,
                     gamma, beta)
